# baseline (speedup 1.0000x reference)
import jax
import jax.numpy as jnp
from jax.experimental import pallas as pl
from jax.experimental.pallas import tpu as pltpu
from jax.experimental.pallas import tpu_sc as plsc

SMALL = 1e-06
_HI = jax.lax.Precision.DEFAULT
_TDN = (((0,), (0,)), ((), ()))


def _pass1_body(z_ref, b_ref, mv_ref, rs_ref, wm_ref, wsm_ref, bsm_ref,
                wval_ref, bval_ref, wa2_ref, ba2_ref,
                popreq_ref, g_ref, nmv_ref, nrs_ref, nwm_ref,
                tab_ref, tp_ref, val_ref, sm_ref):
    j = pl.program_id(0)
    nb = pl.num_programs(0)
    B, M, E = mv_ref.shape
    Nb = z_ref.shape[0]

    @pl.when(j == 0)
    def _init():
        att2 = jnp.sum(mv_ref[...] * wa2_ref[...], axis=2) + ba2_ref[0, 0]
        biastab = jnp.where(rs_ref[...] < SMALL, -1e9, 0.0)
        tab_ref[...] = jnp.concatenate([att2, biastab], axis=1)
        tp_ref[...] = jnp.zeros_like(tp_ref)
        val_ref[...] = jnp.zeros_like(val_ref)
        sm_ref[...] = jnp.zeros_like(sm_ref)

    zb = z_ref[...]
    bb = b_ref[...]
    iota = jax.lax.broadcasted_iota(jnp.int32, (Nb, B), 1)
    onehot = (bb == iota).astype(jnp.float32)

    gath = jax.lax.dot(onehot, tab_ref[...], precision=_HI)
    zws = jax.lax.dot(zb, wsm_ref[...], precision=_HI) + bsm_ref[...]
    pop_s = jax.nn.sigmoid(zws[:, 1:2])
    att1 = zws[:, 2:3]

    logits = att1 + gath[:, :M]
    lr = jnp.where(logits >= 0, logits, 0.01 * logits)
    ml = lr + gath[:, M:]
    mx = jnp.max(ml, axis=1, keepdims=True)
    ex = jnp.exp(ml - mx)
    coefs = ex / jnp.sum(ex, axis=1, keepdims=True)
    popreq = pop_s * coefs
    popreq_ref[...] = popreq

    tp_ref[...] += jax.lax.dot_general(onehot, popreq, _TDN, precision=_HI)
    nodeval = jax.lax.dot(zb, wval_ref[...], precision=_HI) + bval_ref[...]
    val_ref[...] += jax.lax.dot_general(onehot, nodeval, _TDN, precision=_HI)
    sm_ref[...] += jax.lax.dot_general(onehot, zws, _TDN, precision=_HI)

    @pl.when(j == nb - 1)
    def _finalize():
        push_str = jax.nn.sigmoid(sm_ref[:, 0:1])
        wv = jnp.tanh(val_ref[...])
        tp = tp_ref[...]
        rs = rs_ref[...]
        wm = wm_ref[...]
        recp = 1.0 / (tp + SMALL)
        pg = jnp.minimum(rs, tp)
        g_ref[...] = jnp.concatenate([recp * pg, jnp.zeros_like(pg)], axis=1)
        nrs0 = rs - pg
        nrs1 = jnp.where(nrs0 > SMALL, nrs0, 0.0)
        nrs_ref[...] = nrs1 + push_str * wm
        nwm_ref[...] = jnp.concatenate([wm[:, M - 1:M], wm[:, :M - 1]], axis=1)
        nmv_ref[...] = mv_ref[...] + wv[:, None, :] * wm[:, :, None]


def _pass2_body(lohi_ref, pr_ref, b_ref, gg_ref, mv_ref, wout_ref, bout_ref,
                out_ref):
    j = pl.program_id(0)
    nb = pl.num_programs(0)
    B, M, E = mv_ref.shape
    Nb = pr_ref.shape[0]

    bb = b_ref[...]
    oc = pr_ref[...] * gg_ref[:, :M]

    lo = lohi_ref[j]
    hi = lohi_ref[nb + j]

    def body(b, acc):
        mvb = mv_ref[pl.ds(b, 1)][0]
        mask = (bb == b).astype(jnp.float32)
        return acc + jax.lax.dot(oc * mask, mvb, precision=_HI)

    acc = jax.lax.fori_loop(lo, hi + 1, body, jnp.zeros((Nb, E), jnp.float32))
    out_ref[...] = jax.lax.dot(acc, wout_ref[...], precision=_HI) + bout_ref[...]


def kernel(z, batch, memory_values, read_strengths, write_mask,
           Wpush, bpush, Wpop, bpop, Wval, bval, Wout, bout,
           Wa1, ba1, Wa2, ba2):
    N, D = z.shape
    B, M, E = memory_values.shape
    F = Wout.shape[1]
    Nb = 2000
    NB = -(-N // Nb)
    Npad = NB * Nb

    z_p = jnp.pad(z, ((0, Npad - N), (0, 0)))
    batch_p = jnp.concatenate(
        [batch.astype(jnp.int32), jnp.full((Npad - N,), B, jnp.int32)]
    ).reshape(Npad, 1)

    pad_cols = 128 - 3
    wsmall = jnp.concatenate(
        [Wpush, Wpop, Wa1, jnp.zeros((D, pad_cols), jnp.float32)], axis=1)
    bsmall = jnp.concatenate(
        [bpush, bpop, ba1, jnp.zeros((pad_cols,), jnp.float32)]).reshape(1, 128)
    bval2 = bval.reshape(1, E)
    wa2r = Wa2.reshape(1, 1, E)
    ba2r = ba2.reshape(1, 1)
    bout2 = bout.reshape(1, F)

    starts = jnp.arange(NB, dtype=jnp.int32) * Nb
    ends = jnp.minimum(starts + (Nb - 1), N - 1)
    lohi = jnp.concatenate([batch[starts], batch[ends]]).astype(jnp.int32)

    popreq, G, nmv, nrs, nwm = pl.pallas_call(
        _pass1_body,
        grid=(NB,),
        in_specs=[
            pl.BlockSpec((Nb, D), lambda j: (j, 0)),
            pl.BlockSpec((Nb, 1), lambda j: (j, 0)),
            pl.BlockSpec((B, M, E), lambda j: (0, 0, 0)),
            pl.BlockSpec((B, M), lambda j: (0, 0)),
            pl.BlockSpec((B, M), lambda j: (0, 0)),
            pl.BlockSpec((D, 128), lambda j: (0, 0)),
            pl.BlockSpec((1, 128), lambda j: (0, 0)),
            pl.BlockSpec((D, E), lambda j: (0, 0)),
            pl.BlockSpec((1, E), lambda j: (0, 0)),
            pl.BlockSpec((1, 1, E), lambda j: (0, 0, 0)),
            pl.BlockSpec((1, 1), lambda j: (0, 0)),
        ],
        out_specs=[
            pl.BlockSpec((Nb, M), lambda j: (j, 0)),
            pl.BlockSpec((B, 2 * M), lambda j: (0, 0)),
            pl.BlockSpec((B, M, E), lambda j: (0, 0, 0)),
            pl.BlockSpec((B, M), lambda j: (0, 0)),
            pl.BlockSpec((B, M), lambda j: (0, 0)),
        ],
        out_shape=[
            jax.ShapeDtypeStruct((Npad, M), jnp.float32),
            jax.ShapeDtypeStruct((B, 2 * M), jnp.float32),
            jax.ShapeDtypeStruct((B, M, E), jnp.float32),
            jax.ShapeDtypeStruct((B, M), jnp.float32),
            jax.ShapeDtypeStruct((B, M), jnp.float32),
        ],
        scratch_shapes=[
            pltpu.VMEM((B, 2 * M), jnp.float32),
            pltpu.VMEM((B, M), jnp.float32),
            pltpu.VMEM((B, E), jnp.float32),
            pltpu.VMEM((B, 128), jnp.float32),
        ],
    )(z_p, batch_p, memory_values, read_strengths, write_mask,
      wsmall, bsmall, Wval, bval2, wa2r, ba2r)

    W = 128
    NP2 = -(-N // W) * W
    idx2 = jnp.pad(batch.astype(jnp.int32), (0, NP2 - N)).reshape(1, NP2)
    vmesh = plsc.VectorSubcoreMesh(core_axis_name="core",
                                   subcore_axis_name="subcore")

    @pl.kernel(out_type=jax.ShapeDtypeStruct((NP2, 2 * M), jnp.float32),
               mesh=vmesh)
    def _sc_gather(g_hbm, i_hbm, o_hbm):
        def body(i_vmem, o_vmem):
            pltpu.sync_copy(g_hbm.at[i_vmem.at[0]], o_vmem)

        pltpu.emit_pipeline(
            body,
            grid=(NP2 // W,),
            in_specs=[pl.BlockSpec((1, W), lambda i: (0, i))],
            out_specs=[pl.BlockSpec((W, 2 * M), lambda i: (i, 0))],
            core_axis_name=("core", "subcore"),
            dimension_semantics=(pltpu.PARALLEL,),
        )(i_hbm, o_hbm)

    gg = _sc_gather(G, idx2)

    out_p = pl.pallas_call(
        _pass2_body,
        grid_spec=pltpu.PrefetchScalarGridSpec(
            num_scalar_prefetch=1,
            grid=(NB,),
            in_specs=[
                pl.BlockSpec((Nb, M), lambda j, s: (j, 0)),
                pl.BlockSpec((Nb, 1), lambda j, s: (j, 0)),
                pl.BlockSpec((Nb, 2 * M), lambda j, s: (j, 0)),
                pl.BlockSpec((B, M, E), lambda j, s: (0, 0, 0)),
                pl.BlockSpec((E, F), lambda j, s: (0, 0)),
                pl.BlockSpec((1, F), lambda j, s: (0, 0)),
            ],
            out_specs=pl.BlockSpec((Nb, F), lambda j, s: (j, 0)),
        ),
        out_shape=jax.ShapeDtypeStruct((Npad, F), jnp.float32),
    )(lohi, popreq, batch_p, gg, memory_values, Wout, bout2)

    output = out_p[:N]
    message_recepients = jnp.arange(N)
    return (output, message_recepients, nmv, nrs, nwm)


# speedup vs baseline: 1.6561x; 1.6561x over previous
import jax
import jax.numpy as jnp
from jax.experimental import pallas as pl
from jax.experimental.pallas import tpu as pltpu

SMALL = 1e-06
_TDN = (((0,), (0,)), ((), ()))


def _fused_body(lohi_ref, z_ref, b_ref, mv_ref, rs_ref, wm_ref, wsm_ref,
                bsm_ref, wval_ref, bval_ref, wa2_ref, ba2_ref, wout_ref,
                bout_ref,
                out_ref, nmv_ref, nrs_ref, nwm_ref,
                pr_ref, g_ref, tab_ref, tp_ref, val_ref, sm_ref):
    p = pl.program_id(0)
    j = pl.program_id(1)
    nb = pl.num_programs(1)
    B, M, E = mv_ref.shape
    Nb = z_ref.shape[0]

    bb = b_ref[...]
    iota = jax.lax.broadcasted_iota(jnp.int32, (Nb, B), 1)
    onehot = (bb == iota).astype(jnp.float32)

    @pl.when((p == 0) & (j == 0))
    def _init():
        att2 = jnp.sum(mv_ref[...] * wa2_ref[...], axis=2) + ba2_ref[0, 0]
        biastab = jnp.where(rs_ref[...] < SMALL, -1e9, 0.0)
        tab_ref[...] = jnp.concatenate([att2, biastab], axis=1)
        tp_ref[...] = jnp.zeros_like(tp_ref)
        val_ref[...] = jnp.zeros_like(val_ref)
        sm_ref[...] = jnp.zeros_like(sm_ref)

    @pl.when(p == 0)
    def _phase0():
        zb = z_ref[...]
        gath = jax.lax.dot(onehot, tab_ref[...])
        zws = jax.lax.dot(zb, wsm_ref[...]) + bsm_ref[...]
        pop_s = jax.nn.sigmoid(zws[:, 1:2])
        att1 = zws[:, 2:3]

        logits = att1 + gath[:, :M]
        lr = jnp.where(logits >= 0, logits, 0.01 * logits)
        ml = lr + gath[:, M:]
        mx = jnp.max(ml, axis=1, keepdims=True)
        ex = jnp.exp(ml - mx)
        coefs = ex / jnp.sum(ex, axis=1, keepdims=True)
        popreq = pop_s * coefs
        pr_ref[pl.ds(j * Nb, Nb)] = popreq

        tp_ref[...] += jax.lax.dot_general(onehot, popreq, _TDN)
        nodeval = jax.lax.dot(zb, wval_ref[...]) + bval_ref[...]
        val_ref[...] += jax.lax.dot_general(onehot, nodeval, _TDN)
        sm_ref[...] += jax.lax.dot_general(onehot, zws, _TDN)

        @pl.when(j == nb - 1)
        def _finalize():
            push_str = jax.nn.sigmoid(sm_ref[:, 0:1])
            wv = jnp.tanh(val_ref[...])
            tp = tp_ref[...]
            rs = rs_ref[...]
            wm = wm_ref[...]
            recp = 1.0 / (tp + SMALL)
            pg = jnp.minimum(rs, tp)
            g_ref[...] = recp * pg
            nrs0 = rs - pg
            nrs1 = jnp.where(nrs0 > SMALL, nrs0, 0.0)
            nrs_ref[...] = nrs1 + push_str * wm
            nwm_ref[...] = jnp.concatenate(
                [wm[:, M - 1:M], wm[:, :M - 1]], axis=1)
            nmv_ref[...] = mv_ref[...] + wv[:, None, :] * wm[:, :, None]

    @pl.when(p == 1)
    def _phase1():
        oc = pr_ref[pl.ds(j * Nb, Nb)] * jax.lax.dot(onehot, g_ref[...])

        lo = lohi_ref[j]
        hi = lohi_ref[nb + j]

        def body(b, acc):
            mvb = mv_ref[pl.ds(b, 1)][0]
            mask = (bb == b).astype(jnp.float32)
            return acc + jax.lax.dot(oc * mask, mvb)

        acc = jax.lax.fori_loop(lo, hi + 1, body,
                                jnp.zeros((Nb, E), jnp.float32))
        out_ref[...] = jax.lax.dot(acc, wout_ref[...]) + bout_ref[...]


def kernel(z, batch, memory_values, read_strengths, write_mask,
           Wpush, bpush, Wpop, bpop, Wval, bval, Wout, bout,
           Wa1, ba1, Wa2, ba2):
    N, D = z.shape
    B, M, E = memory_values.shape
    F = Wout.shape[1]
    Nb = 2000
    NB = -(-N // Nb)
    Npad = NB * Nb

    z_p = jnp.pad(z, ((0, Npad - N), (0, 0)))
    batch_p = jnp.concatenate(
        [batch.astype(jnp.int32), jnp.full((Npad - N,), B, jnp.int32)]
    ).reshape(Npad, 1)

    pad_cols = 128 - 3
    wsmall = jnp.concatenate(
        [Wpush, Wpop, Wa1, jnp.zeros((D, pad_cols), jnp.float32)], axis=1)
    bsmall = jnp.concatenate(
        [bpush, bpop, ba1, jnp.zeros((pad_cols,), jnp.float32)]).reshape(1, 128)
    bval2 = bval.reshape(1, E)
    wa2r = Wa2.reshape(1, 1, E)
    ba2r = ba2.reshape(1, 1)
    bout2 = bout.reshape(1, F)

    starts = jnp.arange(NB, dtype=jnp.int32) * Nb
    ends = jnp.minimum(starts + (Nb - 1), N - 1)
    lohi = jnp.concatenate([batch[starts], batch[ends]]).astype(jnp.int32)

    zmax = NB - 1

    out_p, nmv, nrs, nwm = pl.pallas_call(
        _fused_body,
        grid_spec=pltpu.PrefetchScalarGridSpec(
            num_scalar_prefetch=1,
            grid=(2, NB),
            in_specs=[
                pl.BlockSpec((Nb, D),
                             lambda p, j, s: (jnp.where(p == 0, j, zmax), 0)),
                pl.BlockSpec((Nb, 1), lambda p, j, s: (j, 0)),
                pl.BlockSpec((B, M, E), lambda p, j, s: (0, 0, 0)),
                pl.BlockSpec((B, M), lambda p, j, s: (0, 0)),
                pl.BlockSpec((B, M), lambda p, j, s: (0, 0)),
                pl.BlockSpec((D, 128), lambda p, j, s: (0, 0)),
                pl.BlockSpec((1, 128), lambda p, j, s: (0, 0)),
                pl.BlockSpec((D, E), lambda p, j, s: (0, 0)),
                pl.BlockSpec((1, E), lambda p, j, s: (0, 0)),
                pl.BlockSpec((1, 1, E), lambda p, j, s: (0, 0, 0)),
                pl.BlockSpec((1, 1), lambda p, j, s: (0, 0)),
                pl.BlockSpec((E, F), lambda p, j, s: (0, 0)),
                pl.BlockSpec((1, F), lambda p, j, s: (0, 0)),
            ],
            out_specs=[
                pl.BlockSpec((Nb, F),
                             lambda p, j, s: (jnp.where(p == 0, 0, j), 0)),
                pl.BlockSpec((B, M, E), lambda p, j, s: (0, 0, 0)),
                pl.BlockSpec((B, M), lambda p, j, s: (0, 0)),
                pl.BlockSpec((B, M), lambda p, j, s: (0, 0)),
            ],
            scratch_shapes=[
                pltpu.VMEM((Npad, M), jnp.float32),
                pltpu.VMEM((B, M), jnp.float32),
                pltpu.VMEM((B, 2 * M), jnp.float32),
                pltpu.VMEM((B, M), jnp.float32),
                pltpu.VMEM((B, E), jnp.float32),
                pltpu.VMEM((B, 128), jnp.float32),
            ],
        ),
        out_shape=[
            jax.ShapeDtypeStruct((Npad, F), jnp.float32),
            jax.ShapeDtypeStruct((B, M, E), jnp.float32),
            jax.ShapeDtypeStruct((B, M), jnp.float32),
            jax.ShapeDtypeStruct((B, M), jnp.float32),
        ],
    )(lohi, z_p, batch_p, memory_values, read_strengths, write_mask,
      wsmall, bsmall, Wval, bval2, wa2r, ba2r, Wout, bout2)

    output = out_p[:N]
    message_recepients = jnp.arange(N)
    return (output, message_recepients, nmv, nrs, nwm)


# speedup vs baseline: 1.7208x; 1.0391x over previous
import jax
import jax.numpy as jnp
from jax.experimental import pallas as pl
from jax.experimental.pallas import tpu as pltpu

SMALL = 1e-06
_HI = jax.lax.Precision.DEFAULT
_TDN = (((0,), (0,)), ((), ()))


def _pass1_body(z_ref, b_ref, mv_ref, rs_ref, wm_ref, wsm_ref, bsm_ref,
                wval_ref, bval_ref, wa2_ref, ba2_ref,
                popreq_ref, g_ref, nmv_ref, nrs_ref, nwm_ref,
                tab_ref, tp_ref, val_ref, sm_ref):
    j = pl.program_id(0)
    nb = pl.num_programs(0)
    B, M, E = mv_ref.shape
    Nb = z_ref.shape[0]

    @pl.when(j == 0)
    def _init():
        att2 = jnp.sum(mv_ref[...] * wa2_ref[...], axis=2) + ba2_ref[0, 0]
        biastab = jnp.where(rs_ref[...] < SMALL, -1e9, 0.0)
        tab_ref[...] = jnp.concatenate([att2, biastab], axis=1)
        tp_ref[...] = jnp.zeros_like(tp_ref)
        val_ref[...] = jnp.zeros_like(val_ref)
        sm_ref[...] = jnp.zeros_like(sm_ref)

    zb = z_ref[...]
    bb = b_ref[...]
    iota = jax.lax.broadcasted_iota(jnp.int32, (Nb, B), 1)
    onehot = (bb == iota).astype(jnp.float32)

    gath = jax.lax.dot(onehot, tab_ref[...], precision=_HI)
    zws = jax.lax.dot(zb, wsm_ref[...], precision=_HI) + bsm_ref[...]
    pop_s = jax.nn.sigmoid(zws[:, 1:2])
    att1 = zws[:, 2:3]

    logits = att1 + gath[:, :M]
    lr = jnp.where(logits >= 0, logits, 0.01 * logits)
    ml = lr + gath[:, M:]
    mx = jnp.max(ml, axis=1, keepdims=True)
    ex = jnp.exp(ml - mx)
    coefs = ex / jnp.sum(ex, axis=1, keepdims=True)
    popreq = pop_s * coefs
    popreq_ref[...] = popreq

    tp_ref[...] += jax.lax.dot_general(onehot, popreq, _TDN, precision=_HI)
    nodeval = jax.lax.dot(zb, wval_ref[...], precision=_HI) + bval_ref[...]
    val_ref[...] += jax.lax.dot_general(onehot, nodeval, _TDN, precision=_HI)
    sm_ref[...] += jax.lax.dot_general(onehot, zws, _TDN, precision=_HI)

    @pl.when(j == nb - 1)
    def _finalize():
        push_str = jax.nn.sigmoid(sm_ref[:, 0:1])
        wv = jnp.tanh(val_ref[...])
        tp = tp_ref[...]
        rs = rs_ref[...]
        wm = wm_ref[...]
        recp = 1.0 / (tp + SMALL)
        pg = jnp.minimum(rs, tp)
        g_ref[...] = recp * pg
        nrs0 = rs - pg
        nrs1 = jnp.where(nrs0 > SMALL, nrs0, 0.0)
        nrs_ref[...] = nrs1 + push_str * wm
        nwm_ref[...] = jnp.concatenate([wm[:, M - 1:M], wm[:, :M - 1]], axis=1)
        nmv_ref[...] = mv_ref[...] + wv[:, None, :] * wm[:, :, None]


def _pass2_body(lohi_ref, pr_ref, b_ref, g_ref, mv_ref, wout_ref, bout_ref,
                out_ref):
    j = pl.program_id(0)
    nb = pl.num_programs(0)
    B, M, E = mv_ref.shape
    Nb = pr_ref.shape[0]

    bb = b_ref[...]
    iota = jax.lax.broadcasted_iota(jnp.int32, (Nb, B), 1)
    onehot = (bb == iota).astype(jnp.float32)
    oc = pr_ref[...] * jax.lax.dot(onehot, g_ref[...], precision=_HI)

    lo = lohi_ref[j]
    hi = lohi_ref[nb + j]

    def body(b, acc):
        mvb = mv_ref[pl.ds(b, 1)][0]
        mask = (bb == b).astype(jnp.float32)
        return acc + jax.lax.dot(oc * mask, mvb, precision=_HI)

    acc = jax.lax.fori_loop(lo, hi + 1, body, jnp.zeros((Nb, E), jnp.float32))
    out_ref[...] = jax.lax.dot(acc, wout_ref[...], precision=_HI) + bout_ref[...]


def kernel(z, batch, memory_values, read_strengths, write_mask,
           Wpush, bpush, Wpop, bpop, Wval, bval, Wout, bout,
           Wa1, ba1, Wa2, ba2):
    N, D = z.shape
    B, M, E = memory_values.shape
    F = Wout.shape[1]
    Nb = 2000
    NB = -(-N // Nb)
    Npad = NB * Nb

    z_p = jnp.pad(z, ((0, Npad - N), (0, 0)))
    batch_p = jnp.concatenate(
        [batch.astype(jnp.int32), jnp.full((Npad - N,), B, jnp.int32)]
    ).reshape(Npad, 1)

    pad_cols = 128 - 3
    wsmall = jnp.concatenate(
        [Wpush, Wpop, Wa1, jnp.zeros((D, pad_cols), jnp.float32)], axis=1)
    bsmall = jnp.concatenate(
        [bpush, bpop, ba1, jnp.zeros((pad_cols,), jnp.float32)]).reshape(1, 128)
    bval2 = bval.reshape(1, E)
    wa2r = Wa2.reshape(1, 1, E)
    ba2r = ba2.reshape(1, 1)
    bout2 = bout.reshape(1, F)

    starts = jnp.arange(NB, dtype=jnp.int32) * Nb
    ends = jnp.minimum(starts + (Nb - 1), N - 1)
    lohi = jnp.concatenate([batch[starts], batch[ends]]).astype(jnp.int32)

    popreq, G, nmv, nrs, nwm = pl.pallas_call(
        _pass1_body,
        grid=(NB,),
        in_specs=[
            pl.BlockSpec((Nb, D), lambda j: (j, 0)),
            pl.BlockSpec((Nb, 1), lambda j: (j, 0)),
            pl.BlockSpec((B, M, E), lambda j: (0, 0, 0)),
            pl.BlockSpec((B, M), lambda j: (0, 0)),
            pl.BlockSpec((B, M), lambda j: (0, 0)),
            pl.BlockSpec((D, 128), lambda j: (0, 0)),
            pl.BlockSpec((1, 128), lambda j: (0, 0)),
            pl.BlockSpec((D, E), lambda j: (0, 0)),
            pl.BlockSpec((1, E), lambda j: (0, 0)),
            pl.BlockSpec((1, 1, E), lambda j: (0, 0, 0)),
            pl.BlockSpec((1, 1), lambda j: (0, 0)),
        ],
        out_specs=[
            pl.BlockSpec((Nb, M), lambda j: (j, 0)),
            pl.BlockSpec((B, M), lambda j: (0, 0)),
            pl.BlockSpec((B, M, E), lambda j: (0, 0, 0)),
            pl.BlockSpec((B, M), lambda j: (0, 0)),
            pl.BlockSpec((B, M), lambda j: (0, 0)),
        ],
        out_shape=[
            jax.ShapeDtypeStruct((Npad, M), jnp.float32),
            jax.ShapeDtypeStruct((B, M), jnp.float32),
            jax.ShapeDtypeStruct((B, M, E), jnp.float32),
            jax.ShapeDtypeStruct((B, M), jnp.float32),
            jax.ShapeDtypeStruct((B, M), jnp.float32),
        ],
        scratch_shapes=[
            pltpu.VMEM((B, 2 * M), jnp.float32),
            pltpu.VMEM((B, M), jnp.float32),
            pltpu.VMEM((B, E), jnp.float32),
            pltpu.VMEM((B, 128), jnp.float32),
        ],
    )(z_p, batch_p, memory_values, read_strengths, write_mask,
      wsmall, bsmall, Wval, bval2, wa2r, ba2r)

    out_p = pl.pallas_call(
        _pass2_body,
        grid_spec=pltpu.PrefetchScalarGridSpec(
            num_scalar_prefetch=1,
            grid=(NB,),
            in_specs=[
                pl.BlockSpec((Nb, M), lambda j, s: (j, 0)),
                pl.BlockSpec((Nb, 1), lambda j, s: (j, 0)),
                pl.BlockSpec((B, M), lambda j, s: (0, 0)),
                pl.BlockSpec((B, M, E), lambda j, s: (0, 0, 0)),
                pl.BlockSpec((E, F), lambda j, s: (0, 0)),
                pl.BlockSpec((1, F), lambda j, s: (0, 0)),
            ],
            out_specs=pl.BlockSpec((Nb, F), lambda j, s: (j, 0)),
        ),
        out_shape=jax.ShapeDtypeStruct((Npad, F), jnp.float32),
    )(lohi, popreq, batch_p, G, memory_values, Wout, bout2)

    output = out_p[:N]
    message_recepients = jnp.arange(N)
    return (output, message_recepients, nmv, nrs, nwm)
